# Initial kernel scaffold; baseline (speedup 1.0000x reference)
import jax
import jax.numpy as jnp
from jax.experimental import pallas as pl


def kernel(node_emb, edge_index, relation_emb, src_w1, src_b1, src_w2, src_b2, dst_w1, dst_b1, dst_w2, dst_b2, edge_w1, edge_b1, edge_w2, edge_b2):
    raise NotImplementedError("write your pallas kernel here")


# speedup vs baseline: 3.2711x; 3.2711x over previous
import functools

import jax
import jax.numpy as jnp
from jax import lax
from jax.experimental import pallas as pl
from jax.experimental.pallas import tpu as pltpu
from jax.experimental.pallas import tpu_sc as plsc

TEMPERATURE = 0.5
BIAS = 0.0001

NC = 2
NS = 16
NW = NC * NS
LANES = 16


def _node_mlp_body(x_ref, w1_ref, b1_ref, w2_ref, b2_ref, o_ref):
    h = jnp.dot(x_ref[...], w1_ref[...], preferred_element_type=jnp.float32)
    h = jnp.maximum(h + b1_ref[...], 0.0)
    o_ref[...] = (
        jnp.dot(h, w2_ref[...], preferred_element_type=jnp.float32)
        + b2_ref[...]
    )


def _edge_mlp_body(x_ref, u_ref, w1_ref, b1_ref, w2_ref, b2_ref, o_ref):
    h = jnp.dot(x_ref[...], w1_ref[...], preferred_element_type=jnp.float32)
    h = jnp.maximum(h + b1_ref[...], 0.0)
    e = jnp.dot(h, w2_ref[...], preferred_element_type=jnp.float32) + b2_ref[...]
    u = u_ref[...]
    eps = (BIAS - (1.0 - BIAS)) * u + (1.0 - BIAS)
    g = jnp.log(eps) - jnp.log(1.0 - eps)
    o_ref[...] = e + g


def _node_scores(node_emb, w1c, b1c, w2c, b2c):
    n, d = node_emb.shape
    h2 = w1c.shape[1]
    blk = 2000
    grid = (n // blk,)
    return pl.pallas_call(
        _node_mlp_body,
        grid=grid,
        in_specs=[
            pl.BlockSpec((blk, d), lambda i: (i, 0)),
            pl.BlockSpec((d, h2), lambda i: (0, 0)),
            pl.BlockSpec((1, h2), lambda i: (0, 0)),
            pl.BlockSpec((h2, 2), lambda i: (0, 0)),
            pl.BlockSpec((1, 2), lambda i: (0, 0)),
        ],
        out_specs=pl.BlockSpec((blk, 2), lambda i: (i, 0)),
        out_shape=jax.ShapeDtypeStruct((n, 2), jnp.float32),
    )(node_emb, w1c, b1c, w2c, b2c)


def _edge_gate(relation_emb, u2, w1, b1, w2, b2):
    e, de = relation_emb.shape
    h = w1.shape[1]
    blk = 10000
    grid = (e // blk,)
    return pl.pallas_call(
        _edge_mlp_body,
        grid=grid,
        in_specs=[
            pl.BlockSpec((blk, de), lambda i: (i, 0)),
            pl.BlockSpec((blk, 1), lambda i: (i, 0)),
            pl.BlockSpec((de, h), lambda i: (0, 0)),
            pl.BlockSpec((1, h), lambda i: (0, 0)),
            pl.BlockSpec((h, 1), lambda i: (0, 0)),
            pl.BlockSpec((1, 1), lambda i: (0, 0)),
        ],
        out_specs=pl.BlockSpec((blk, 1), lambda i: (i, 0)),
        out_shape=jax.ShapeDtypeStruct((e, 1), jnp.float32),
    )(relation_emb, u2, w1, b1, w2, b2)


def _sc_gather_gate(scores, src, dst, ge):
    n2 = scores.shape[0]
    e = src.shape[0]
    ch = e // NW
    full = (ch // LANES) * LANES
    tail = ch - full
    mesh = plsc.VectorSubcoreMesh(
        core_axis_name="c", subcore_axis_name="s",
        num_cores=NC, num_subcores=NS)

    @functools.partial(
        pl.kernel,
        out_type=[
            jax.ShapeDtypeStruct((e,), jnp.float32),
            jax.ShapeDtypeStruct((NW * LANES,), jnp.float32),
        ],
        mesh=mesh,
        compiler_params=pltpu.CompilerParams(needs_layout_passes=False),
        scratch_types=[
            pltpu.VMEM((n2,), jnp.float32),
            pltpu.VMEM((ch,), jnp.int32),
            pltpu.VMEM((ch,), jnp.int32),
            pltpu.VMEM((ch,), jnp.float32),
            pltpu.VMEM((ch,), jnp.float32),
            pltpu.VMEM((LANES,), jnp.float32),
        ],
    )
    def sc_kernel(scores_hbm, src_hbm, dst_hbm, ge_hbm, aug_hbm, part_hbm,
                  table_v, src_v, dst_v, ge_v, aug_v, acc_v):
        c = lax.axis_index("c")
        s = lax.axis_index("s")
        wid = s * NC + c
        base = wid * ch
        pltpu.sync_copy(scores_hbm, table_v)
        pltpu.sync_copy(src_hbm.at[pl.ds(base, ch)], src_v)
        pltpu.sync_copy(dst_hbm.at[pl.ds(base, ch)], dst_v)
        pltpu.sync_copy(ge_hbm.at[pl.ds(base, ch)], ge_v)

        def gate16(off):
            sidx = src_v[pl.ds(off, LANES)]
            didx = dst_v[pl.ds(off, LANES)]
            ws = plsc.load_gather(table_v, [sidx * 2])
            wd = plsc.load_gather(table_v, [didx * 2 + 1])
            x = (ws + wd + ge_v[pl.ds(off, LANES)]) * (1.0 / TEMPERATURE)
            return 1.0 / (1.0 + jnp.exp(-x))

        def body(i, acc):
            off = i * LANES
            a = gate16(off)
            aug_v[pl.ds(off, LANES)] = a
            return acc + a

        acc = lax.fori_loop(0, full // LANES, body,
                            jnp.zeros((LANES,), jnp.float32))
        if tail:
            off = ch - LANES
            a = gate16(off)
            aug_v[pl.ds(off, LANES)] = a
            lane = lax.iota(jnp.int32, LANES)
            acc = acc + jnp.where(lane >= (LANES - tail), a, 0.0)
        acc_v[...] = acc
        pltpu.sync_copy(aug_v, aug_hbm.at[pl.ds(base, ch)])
        pltpu.sync_copy(acc_v, part_hbm.at[pl.ds(wid * LANES, LANES)])

    return sc_kernel(scores, src, dst, ge)


def kernel(node_emb, edge_index, relation_emb,
           src_w1, src_b1, src_w2, src_b2,
           dst_w1, dst_b1, dst_w2, dst_b2,
           edge_w1, edge_b1, edge_w2, edge_b2):
    n, d = node_emb.shape
    e = edge_index.shape[1]
    h = src_w1.shape[1]

    w1c = jnp.concatenate([src_w1, dst_w1], axis=1)
    b1c = jnp.concatenate([src_b1, dst_b1]).reshape(1, 2 * h)
    z = jnp.zeros((h, 1), jnp.float32)
    w2c = jnp.concatenate(
        [jnp.concatenate([src_w2, z], axis=1),
         jnp.concatenate([z, dst_w2], axis=1)], axis=0)
    b2c = jnp.stack([src_b2[0], dst_b2[0]]).reshape(1, 2)

    scores = _node_scores(node_emb, w1c, b1c, w2c, b2c)

    u = jax.random.uniform(jax.random.key(12345), (e,), jnp.float32)
    ge2 = _edge_gate(relation_emb, u.reshape(e, 1),
                     edge_w1, edge_b1.reshape(1, h),
                     edge_w2, edge_b2.reshape(1, 1))

    src = edge_index[0]
    dst = edge_index[1]
    aug, partials = _sc_gather_gate(scores.reshape(2 * n), src, dst,
                                    ge2.reshape(e))

    reg = 1.0 - jnp.sum(partials) / e
    return (reg, aug)
